# baseline (speedup 1.0000x reference)
import functools

import jax
import jax.numpy as jnp
from jax import lax
from jax.experimental import pallas as pl
from jax.experimental.pallas import tpu as pltpu
from jax.experimental.pallas import tpu_sc as plsc

BATCH = 16384
EMBED = 32
ROWW = 33
NVOC = 100000
NC = 2
NS = 16
NW = NC * NS
BPW = BATCH // NW
LANES = 16
CR = 128
CHUNKS = BPW // CR
RSTR = 128


def _sc_body(idx_u_hbm, idx_m_hbm, ucat_hbm, mcat_hbm,
             partials_hbm, bsum_hbm,
             idxu_v, idxm_v, urows0, urows1, mrows0, mrows1, bsum_v, acc_v,
             semu0, semu1, semm0, semm1):
    wid = lax.axis_index("s") * NC + lax.axis_index("c")
    base = wid * BPW
    urows = (urows0, urows1)
    mrows = (mrows0, mrows1)
    semu = (semu0, semu1)
    semm = (semm0, semm1)

    pltpu.sync_copy(idx_u_hbm.at[pl.ds(base, BPW)], idxu_v)
    pltpu.sync_copy(idx_m_hbm.at[pl.ds(base, BPW)], idxm_v)

    def issue_chunk(c, bb):
        def issue(i, carry):
            ivu = idxu_v[pl.ds(c * CR + i * LANES, LANES)]
            ivm = idxm_v[pl.ds(c * CR + i * LANES, LANES)]
            for k in range(LANES):
                r = i * LANES + k
                pltpu.make_async_copy(
                    ucat_hbm.at[pl.ds(ivu[k], 1), :],
                    urows[bb].at[pl.ds(r, 1), :], semu[bb]).start()
                pltpu.make_async_copy(
                    mcat_hbm.at[pl.ds(ivm[k], 1), :],
                    mrows[bb].at[pl.ds(r, 1), :], semm[bb]).start()
            return carry
        lax.fori_loop(0, CR // LANES, issue, 0)

    def drain_chunk(bb):
        pltpu.make_async_copy(
            ucat_hbm.at[pl.ds(0, CR), :], urows[bb], semu[bb]).wait()
        pltpu.make_async_copy(
            mcat_hbm.at[pl.ds(0, CR), :], mrows[bb], semm[bb]).wait()

    riota = lax.iota(jnp.int32, LANES)

    def process_chunk(c, bb, acc):
        u = urows[bb]
        m = mrows[bb]

        def dot_body(j, a):
            row = j // 2
            col = (j % 2) * LANES
            return a + u[row, pl.ds(col, LANES)] * m[row, pl.ds(col, LANES)]
        acc = lax.fori_loop(0, CR * 2, dot_body, acc)

        cvec = jnp.full((LANES,), EMBED, jnp.int32)

        def bias_body(i, carry):
            rvec = i * LANES + riota
            ub = plsc.load_gather(u, [rvec, cvec])
            mb = plsc.load_gather(m, [rvec, cvec])
            bsum_v[pl.ds(c * CR + i * LANES, LANES)] = ub + mb
            return carry
        lax.fori_loop(0, CR // LANES, bias_body, 0)
        return acc

    acc = jnp.zeros((LANES,), jnp.float32)
    for c in range(CHUNKS):
        bb = c % 2
        issue_chunk(c, bb)
        if c >= 1:
            pb = (c - 1) % 2
            drain_chunk(pb)
            acc = process_chunk(c - 1, pb, acc)
    lastb = (CHUNKS - 1) % 2
    drain_chunk(lastb)
    acc = process_chunk(CHUNKS - 1, lastb, acc)

    pltpu.sync_copy(bsum_v, bsum_hbm.at[pl.ds(base, BPW)])
    acc_v[0, pl.ds(0, LANES)] = acc
    pltpu.sync_copy(acc_v, partials_hbm.at[pl.ds(wid, 1), :])


def _sc_stage(idx_u, idx_m, ucat, mcat):
    mesh = plsc.VectorSubcoreMesh(core_axis_name="c", subcore_axis_name="s")
    return pl.kernel(
        _sc_body,
        out_type=(
            jax.ShapeDtypeStruct((NW, LANES), jnp.float32),
            jax.ShapeDtypeStruct((BATCH,), jnp.float32),
        ),
        mesh=mesh,
        scratch_types=[
            pltpu.VMEM((BPW,), jnp.int32),
            pltpu.VMEM((BPW,), jnp.int32),
            pltpu.VMEM((CR, ROWW), jnp.float32),
            pltpu.VMEM((CR, ROWW), jnp.float32),
            pltpu.VMEM((CR, ROWW), jnp.float32),
            pltpu.VMEM((CR, ROWW), jnp.float32),
            pltpu.VMEM((BPW,), jnp.float32),
            pltpu.VMEM((1, LANES), jnp.float32),
            pltpu.SemaphoreType.DMA,
            pltpu.SemaphoreType.DMA,
            pltpu.SemaphoreType.DMA,
            pltpu.SemaphoreType.DMA,
        ],
        compiler_params=pltpu.CompilerParams(needs_layout_passes=False),
    )(idx_u, idx_m, ucat, mcat)


PREPB = 512


def _prep_body(et_ref, b_ref, o_ref):
    t = jnp.transpose(et_ref[...])
    o_ref[...] = jnp.concatenate([t, b_ref[...].reshape(PREPB, 1)], axis=1)


def _prep_table(emb_t, b1d, nrows):
    grid = (nrows + PREPB - 1) // PREPB
    return pl.pallas_call(
        _prep_body,
        grid=(grid,),
        in_specs=[
            pl.BlockSpec((EMBED, PREPB), lambda i: (0, i)),
            pl.BlockSpec((PREPB,), lambda i: (i,)),
        ],
        out_specs=pl.BlockSpec((PREPB, ROWW), lambda i: (i, 0)),
        out_shape=jax.ShapeDtypeStruct((grid * PREPB, ROWW), jnp.float32),
    )(emb_t, b1d)


def _tc_body(p_ref, b_ref, o_ref):
    s = jnp.sum(p_ref[...])
    o_ref[...] = jax.nn.sigmoid(b_ref[...] + s)


def _tc_finish(partials, bsum):
    out = pl.pallas_call(
        _tc_body,
        out_shape=jax.ShapeDtypeStruct((128, 128), jnp.float32),
    )(partials, bsum.reshape(128, 128))
    return out.reshape(BATCH, 1)


@jax.jit
def kernel(inputs, user_embedding, user_bias, movie_embedding, movie_bias):
    idx_u = inputs[:, 0]
    idx_m = inputs[:, 1]
    ucat = _prep_table(user_embedding.T, user_bias.reshape(-1), NVOC)
    mcat = _prep_table(movie_embedding.T, movie_bias.reshape(-1), NVOC)
    partials, bsum = _sc_stage(idx_u, idx_m, ucat, mcat)
    return _tc_finish(partials, bsum)


# speedup vs baseline: 2.5710x; 2.5710x over previous
import functools

import jax
import jax.numpy as jnp
from jax import lax
from jax.experimental import pallas as pl
from jax.experimental.pallas import tpu as pltpu
from jax.experimental.pallas import tpu_sc as plsc

BATCH = 16384
EMBED = 32
ROWW = 33
NVOC = 100000
NC = 2
NS = 16
NW = NC * NS
BPW = BATCH // NW
LANES = 16
CR = 128
CHUNKS = BPW // CR
RSTR = 128


def _sc_body(idx_u_hbm, idx_m_hbm, ucat_hbm, mcat_hbm,
             partials_hbm, bsum_hbm,
             idxu_v, idxm_v, urows0, urows1, mrows0, mrows1, bsum_v, acc_v,
             semu0, semu1, semm0, semm1):
    wid = lax.axis_index("s") * NC + lax.axis_index("c")
    base = wid * BPW
    urows = (urows0, urows1)
    mrows = (mrows0, mrows1)
    semu = (semu0, semu1)
    semm = (semm0, semm1)

    pltpu.sync_copy(idx_u_hbm.at[pl.ds(base, BPW)], idxu_v)
    pltpu.sync_copy(idx_m_hbm.at[pl.ds(base, BPW)], idxm_v)

    def issue_chunk(c, bb):
        def issue(i, carry):
            ivu = idxu_v[pl.ds(c * CR + i * LANES, LANES)]
            ivm = idxm_v[pl.ds(c * CR + i * LANES, LANES)]
            for k in range(LANES):
                r = i * LANES + k
                pltpu.make_async_copy(
                    ucat_hbm.at[pl.ds(ivu[k], 1), :],
                    urows[bb].at[pl.ds(r, 1), :], semu[bb]).start()
                pltpu.make_async_copy(
                    mcat_hbm.at[pl.ds(ivm[k], 1), :],
                    mrows[bb].at[pl.ds(r, 1), :], semm[bb]).start()
            return carry
        lax.fori_loop(0, CR // LANES, issue, 0)

    def drain_chunk(bb):
        pltpu.make_async_copy(
            ucat_hbm.at[pl.ds(0, CR), :], urows[bb], semu[bb]).wait()
        pltpu.make_async_copy(
            mcat_hbm.at[pl.ds(0, CR), :], mrows[bb], semm[bb]).wait()

    riota = lax.iota(jnp.int32, LANES)

    def process_chunk(c, bb, acc):
        u = urows[bb]
        m = mrows[bb]

        def dot_body(j, a):
            row = j // 2
            col = (j % 2) * LANES
            return a + u[row, pl.ds(col, LANES)] * m[row, pl.ds(col, LANES)]
        acc = lax.fori_loop(0, CR * 2, dot_body, acc)

        cvec = jnp.full((LANES,), EMBED, jnp.int32)

        def bias_body(i, carry):
            rvec = i * LANES + riota
            ub = plsc.load_gather(u, [rvec, cvec])
            mb = plsc.load_gather(m, [rvec, cvec])
            bsum_v[pl.ds(c * CR + i * LANES, LANES)] = ub + mb
            return carry
        lax.fori_loop(0, CR // LANES, bias_body, 0)
        return acc

    acc = jnp.zeros((LANES,), jnp.float32)
    for c in range(CHUNKS):
        bb = c % 2
        issue_chunk(c, bb)
        if c >= 1:
            pb = (c - 1) % 2
            drain_chunk(pb)
            acc = process_chunk(c - 1, pb, acc)
    lastb = (CHUNKS - 1) % 2
    drain_chunk(lastb)
    acc = process_chunk(CHUNKS - 1, lastb, acc)

    pltpu.sync_copy(bsum_v, bsum_hbm.at[pl.ds(base, BPW)])
    acc_v[0, pl.ds(0, LANES)] = acc
    pltpu.sync_copy(acc_v, partials_hbm.at[pl.ds(wid, 1), :])


def _sc_stage(idx_u, idx_m, ucat, mcat):
    mesh = plsc.VectorSubcoreMesh(core_axis_name="c", subcore_axis_name="s")
    return pl.kernel(
        _sc_body,
        out_type=(
            jax.ShapeDtypeStruct((NW, LANES), jnp.float32),
            jax.ShapeDtypeStruct((BATCH,), jnp.float32),
        ),
        mesh=mesh,
        scratch_types=[
            pltpu.VMEM((BPW,), jnp.int32),
            pltpu.VMEM((BPW,), jnp.int32),
            pltpu.VMEM((CR, ROWW), jnp.float32),
            pltpu.VMEM((CR, ROWW), jnp.float32),
            pltpu.VMEM((CR, ROWW), jnp.float32),
            pltpu.VMEM((CR, ROWW), jnp.float32),
            pltpu.VMEM((BPW,), jnp.float32),
            pltpu.VMEM((1, LANES), jnp.float32),
            pltpu.SemaphoreType.DMA,
            pltpu.SemaphoreType.DMA,
            pltpu.SemaphoreType.DMA,
            pltpu.SemaphoreType.DMA,
        ],
        compiler_params=pltpu.CompilerParams(needs_layout_passes=False),
    )(idx_u, idx_m, ucat, mcat)


def _tc_body(p_ref, b_ref, o_ref):
    s = jnp.sum(p_ref[...])
    o_ref[...] = jax.nn.sigmoid(b_ref[...] + s)


def _tc_finish(partials, bsum):
    out = pl.pallas_call(
        _tc_body,
        out_shape=jax.ShapeDtypeStruct((128, 128), jnp.float32),
    )(partials, bsum.reshape(128, 128))
    return out.reshape(BATCH, 1)


@jax.jit
def kernel(inputs, user_embedding, user_bias, movie_embedding, movie_bias):
    idx_u = inputs[:, 0]
    idx_m = inputs[:, 1]
    ucat = jnp.concatenate(
        [user_embedding[:NVOC], user_bias[:NVOC]], axis=1)
    mcat = jnp.concatenate([movie_embedding, movie_bias], axis=1)
    partials, bsum = _sc_stage(idx_u, idx_m, ucat, mcat)
    return _tc_finish(partials, bsum)
